# Initial kernel scaffold; baseline (speedup 1.0000x reference)
import jax
import jax.numpy as jnp
from jax.experimental import pallas as pl


def kernel(input_x, table):
    raise NotImplementedError("write your pallas kernel here")


# speedup vs baseline: 2.9557x; 2.9557x over previous
import functools

import jax
import jax.numpy as jnp
from jax import lax
from jax.experimental import pallas as pl
from jax.experimental.pallas import tpu as pltpu
from jax.experimental.pallas import tpu_sc as plsc

BATCH = 16384
N_FIELDS = 26
EMBED_DIM = 128
TOTAL = BATCH * N_FIELDS

NC = 2
NS = 16
NW = NC * NS
CHUNK = TOTAL // NW
ROWS_PER_DMA = 128
N_DMAS = CHUNK // ROWS_PER_DMA
VECS = CHUNK // 16


def _make_kernel():
    mesh = plsc.VectorSubcoreMesh(core_axis_name="c", subcore_axis_name="s")

    @functools.partial(
        pl.kernel,
        mesh=mesh,
        out_type=jax.ShapeDtypeStruct((TOTAL, EMBED_DIM), jnp.float32),
        scratch_types=[
            pltpu.VMEM((CHUNK,), jnp.int32),
            pltpu.VMEM((ROWS_PER_DMA, EMBED_DIM), jnp.float32),
            pltpu.SemaphoreType.DMA,
        ],
    )
    def k(x_hbm, table_hbm, out_hbm, idx_v, rows_v, sem):
        wid = lax.axis_index("s") * NC + lax.axis_index("c")
        base = wid * CHUNK

        pltpu.sync_copy(x_hbm.at[pl.ds(base, CHUNK)], idx_v)

        lane = lax.iota(jnp.int32, 16)

        def add_off(i, carry):
            pos = i * 16 + lane
            off = (pos % N_FIELDS) * 1000
            idx_v[pl.ds(i * 16, 16)] = idx_v[pl.ds(i * 16, 16)] + off
            return carry

        lax.fori_loop(0, VECS, add_off, 0)

        def body(j, carry):
            idx_slice = idx_v.at[pl.ds(j * ROWS_PER_DMA, ROWS_PER_DMA)]
            pltpu.async_copy(table_hbm.at[idx_slice], rows_v, sem).wait()
            pltpu.sync_copy(
                rows_v, out_hbm.at[pl.ds(base + j * ROWS_PER_DMA, ROWS_PER_DMA)]
            )
            return carry

        lax.fori_loop(0, N_DMAS, body, 0)

    return k


_kernel_fn = _make_kernel()


def kernel(input_x, table):
    x = jnp.asarray(input_x, jnp.int32).reshape(-1)
    out = _kernel_fn(x, table)
    return out.reshape(BATCH, N_FIELDS, EMBED_DIM)


# speedup vs baseline: 3.3345x; 1.1282x over previous
import functools

import jax
import jax.numpy as jnp
from jax import lax
from jax.experimental import pallas as pl
from jax.experimental.pallas import tpu as pltpu
from jax.experimental.pallas import tpu_sc as plsc

BATCH = 16384
N_FIELDS = 26
EMBED_DIM = 128
TOTAL = BATCH * N_FIELDS

NC = 2
NS = 16
NW = NC * NS
CHUNK = TOTAL // NW
ROWS = 104
N_DMAS = CHUNK // ROWS
NBUF = 8
LAG = 4
VECS = CHUNK // 16
N_GROUPS = N_DMAS // NBUF


def _make_kernel():
    mesh = plsc.VectorSubcoreMesh(core_axis_name="c", subcore_axis_name="s")

    @functools.partial(
        pl.kernel,
        mesh=mesh,
        out_type=jax.ShapeDtypeStruct((TOTAL, EMBED_DIM), jnp.float32),
        scratch_types=[pltpu.VMEM((CHUNK,), jnp.int32)]
        + [pltpu.VMEM((ROWS, EMBED_DIM), jnp.float32) for _ in range(NBUF)]
        + [pltpu.SemaphoreType.DMA for _ in range(2 * NBUF)],
    )
    def k(x_hbm, table_hbm, out_hbm, idx_v, *rest):
        bufs = rest[:NBUF]
        gsems = rest[NBUF : 2 * NBUF]
        osems = rest[2 * NBUF : 3 * NBUF]

        wid = lax.axis_index("s") * NC + lax.axis_index("c")
        base = wid * CHUNK

        pltpu.sync_copy(x_hbm.at[pl.ds(base, CHUNK)], idx_v)

        lane = lax.iota(jnp.int32, 16)

        def add_off(i, carry):
            pos = i * 16 + lane
            off = (pos % N_FIELDS) * 1000
            idx_v[pl.ds(i * 16, 16)] = idx_v[pl.ds(i * 16, 16)] + off
            return carry

        lax.fori_loop(0, VECS, add_off, 0)

        def g_issue(t, b):
            idx_slice = idx_v.at[pl.ds(t * ROWS, ROWS)]
            pltpu.async_copy(table_hbm.at[idx_slice], bufs[b], gsems[b])

        def g_wait(b):
            pltpu.make_async_copy(
                table_hbm.at[pl.ds(0, ROWS)], bufs[b], gsems[b]
            ).wait()

        def o_issue(j, b):
            pltpu.async_copy(
                bufs[b], out_hbm.at[pl.ds(base + j * ROWS, ROWS)], osems[b]
            )

        def o_wait(b):
            pltpu.make_async_copy(
                bufs[b], out_hbm.at[pl.ds(base, ROWS)], osems[b]
            ).wait()

        for b in range(NBUF):
            g_issue(b, b)
            if b >= LAG:
                j = b - LAG
                g_wait(j % NBUF)
                o_issue(j, j % NBUF)

        def group(g, carry):
            for b in range(NBUF):
                t = g * NBUF + b
                o_wait(b)
                g_issue(t, b)
                j = t - LAG
                bj = (b - LAG) % NBUF
                g_wait(bj)
                o_issue(j, bj)
            return carry

        lax.fori_loop(1, N_GROUPS, group, 0)

        for j in range(N_DMAS - LAG, N_DMAS):
            bj = j % NBUF
            g_wait(bj)
            o_issue(j, bj)
        for b in range(NBUF):
            o_wait(b)

    return k


_kernel_fn = _make_kernel()


def kernel(input_x, table):
    x = jnp.asarray(input_x, jnp.int32).reshape(-1)
    out = _kernel_fn(x, table)
    return out.reshape(BATCH, N_FIELDS, EMBED_DIM)


# speedup vs baseline: 5.5551x; 1.6659x over previous
import functools

import jax
import jax.numpy as jnp
from jax import lax
from jax.experimental import pallas as pl
from jax.experimental.pallas import tpu as pltpu
from jax.experimental.pallas import tpu_sc as plsc

BATCH = 16384
N_FIELDS = 26
EMBED_DIM = 128
TOTAL = BATCH * N_FIELDS

NC = 2
NS = 16
NW = NC * NS
CHUNK = TOTAL // NW
ROWS = 104
N_DMAS = CHUNK // ROWS
NBUF = 8
LAG = 4
VECS = CHUNK // 16
N_GROUPS = N_DMAS // NBUF


def _make_kernel():
    mesh = plsc.VectorSubcoreMesh(core_axis_name="c", subcore_axis_name="s")

    @functools.partial(
        pl.kernel,
        mesh=mesh,
        out_type=jax.ShapeDtypeStruct((BATCH, N_FIELDS, EMBED_DIM), jnp.float32),
        scratch_types=[pltpu.VMEM((CHUNK,), jnp.int32)]
        + [pltpu.VMEM((ROWS, EMBED_DIM), jnp.float32) for _ in range(NBUF)]
        + [pltpu.SemaphoreType.DMA for _ in range(2 * NBUF)],
    )
    def k(x_hbm, table_hbm, out_hbm, idx_v, *rest):
        bufs = rest[:NBUF]
        gsems = rest[NBUF : 2 * NBUF]
        osems = rest[2 * NBUF : 3 * NBUF]

        wid = lax.axis_index("s") * NC + lax.axis_index("c")
        base = wid * CHUNK
        bbase = wid * (CHUNK // N_FIELDS)

        pltpu.sync_copy(x_hbm.at[pl.ds(base, CHUNK)], idx_v)

        lane = lax.iota(jnp.int32, 16)

        def add_off(i, carry):
            pos = i * 16 + lane
            off = (pos % N_FIELDS) * 1000
            idx_v[pl.ds(i * 16, 16)] = idx_v[pl.ds(i * 16, 16)] + off
            return carry

        lax.fori_loop(0, VECS, add_off, 0)

        def g_issue(t, b):
            idx_slice = idx_v.at[pl.ds(t * ROWS, ROWS)]
            pltpu.async_copy(table_hbm.at[idx_slice], bufs[b], gsems[b])

        def g_wait(b):
            pltpu.make_async_copy(
                table_hbm.at[pl.ds(0, ROWS)], bufs[b], gsems[b]
            ).wait()

        def o_issue(j, b):
            for r in range(ROWS // N_FIELDS):
                pltpu.async_copy(
                    bufs[b].at[pl.ds(r * N_FIELDS, N_FIELDS)],
                    out_hbm.at[bbase + j * (ROWS // N_FIELDS) + r],
                    osems[b],
                )

        def o_wait(b):
            for _ in range(ROWS // N_FIELDS):
                pltpu.make_async_copy(
                    bufs[b].at[pl.ds(0, N_FIELDS)], out_hbm.at[0], osems[b]
                ).wait()

        for b in range(NBUF):
            g_issue(b, b)
            if b >= LAG:
                j = b - LAG
                g_wait(j % NBUF)
                o_issue(j, j % NBUF)

        def group(g, carry):
            for b in range(NBUF):
                t = g * NBUF + b
                o_wait(b)
                g_issue(t, b)
                j = t - LAG
                bj = (b - LAG) % NBUF
                g_wait(bj)
                o_issue(j, bj)
            return carry

        lax.fori_loop(1, N_GROUPS, group, 0)

        for j in range(N_DMAS - LAG, N_DMAS):
            bj = j % NBUF
            g_wait(bj)
            o_issue(j, bj)
        for b in range(NBUF):
            o_wait(b)

    return k


_kernel_fn = _make_kernel()


def kernel(input_x, table):
    x = jnp.asarray(input_x, jnp.int32).reshape(-1)
    return _kernel_fn(x, table)


# speedup vs baseline: 10.2258x; 1.8408x over previous
import functools

import jax
import jax.numpy as jnp
from jax import lax
from jax.experimental import pallas as pl
from jax.experimental.pallas import tpu as pltpu
from jax.experimental.pallas import tpu_sc as plsc

BATCH = 16384
N_FIELDS = 26
EMBED_DIM = 128
TOTAL = BATCH * N_FIELDS

NC = 2
NS = 16
NW = NC * NS
B_PER_W = BATCH // NW
CHUNK = B_PER_W * N_FIELDS
ROWS = 128
DMAS_PER_F = B_PER_W // ROWS
N_DMAS = N_FIELDS * DMAS_PER_F
NBUF = 4
LAG = 2
VECS_PER_F = B_PER_W // 16


def _make_kernel():
    mesh = plsc.VectorSubcoreMesh(core_axis_name="c", subcore_axis_name="s")

    @functools.partial(
        pl.kernel,
        mesh=mesh,
        out_type=jax.ShapeDtypeStruct((TOTAL, EMBED_DIM), jnp.float32),
        scratch_types=[pltpu.VMEM((CHUNK,), jnp.int32)]
        + [pltpu.VMEM((ROWS, EMBED_DIM), jnp.float32) for _ in range(NBUF)]
        + [pltpu.SemaphoreType.DMA for _ in range(2 * NBUF + 1)],
    )
    def k(x_hbm, table_hbm, out_hbm, idx_v, *rest):
        bufs = rest[:NBUF]
        gsems = rest[NBUF : 2 * NBUF]
        osems = rest[2 * NBUF : 3 * NBUF]
        isem = rest[3 * NBUF]

        wid = lax.axis_index("s") * NC + lax.axis_index("c")
        b0 = wid * B_PER_W

        for f in range(N_FIELDS):
            pltpu.async_copy(
                x_hbm.at[pl.ds(f * BATCH + b0, B_PER_W)],
                idx_v.at[pl.ds(f * B_PER_W, B_PER_W)],
                isem,
            )
        for f in range(N_FIELDS):
            pltpu.make_async_copy(
                x_hbm.at[pl.ds(0, B_PER_W)],
                idx_v.at[pl.ds(0, B_PER_W)],
                isem,
            ).wait()

        def add_off(f, carry):
            def inner(g, carry2):
                s = f * B_PER_W + g * 16
                idx_v[pl.ds(s, 16)] = idx_v[pl.ds(s, 16)] + f * 1000
                return carry2

            return lax.fori_loop(0, VECS_PER_F, inner, carry)

        lax.fori_loop(0, N_FIELDS, add_off, 0)

        def g_issue(t, b):
            idx_slice = idx_v.at[pl.ds(t * ROWS, ROWS)]
            pltpu.async_copy(table_hbm.at[idx_slice], bufs[b], gsems[b])

        def g_wait(b):
            pltpu.make_async_copy(
                table_hbm.at[pl.ds(0, ROWS)], bufs[b], gsems[b]
            ).wait()

        def o_issue(j, b):
            f = j // DMAS_PER_F
            c = j % DMAS_PER_F
            dst = out_hbm.at[pl.ds(f * BATCH + b0 + c * ROWS, ROWS)]
            pltpu.async_copy(bufs[b], dst, osems[b])

        def o_wait(b):
            pltpu.make_async_copy(
                bufs[b], out_hbm.at[pl.ds(0, ROWS)], osems[b]
            ).wait()

        for b in range(NBUF):
            g_issue(b, b)
            if b >= LAG:
                j = b - LAG
                g_wait(j % NBUF)
                o_issue(j, j % NBUF)

        def group(g, carry):
            for b in range(NBUF):
                t = g * NBUF + b
                o_wait(b)
                g_issue(t, b)
                j = t - LAG
                bj = (b - LAG) % NBUF
                g_wait(bj)
                o_issue(j, bj)
            return carry

        lax.fori_loop(1, N_DMAS // NBUF, group, 0)

        for j in range(N_DMAS - LAG, N_DMAS):
            g_wait(j % NBUF)
            o_issue(j, j % NBUF)
        for b in range(NBUF):
            o_wait(b)

    return k


_kernel_fn = _make_kernel()


def kernel(input_x, table):
    xq = jnp.transpose(input_x).reshape(-1).astype(jnp.int32)
    out = _kernel_fn(xq, table)
    return jnp.transpose(
        out.reshape(N_FIELDS, BATCH, EMBED_DIM), (1, 0, 2)
    )
